# baseline (speedup 1.0000x reference)
import jax
import jax.numpy as jnp
from jax.experimental import pallas as pl

B = 2
C = 96
EN = 8
HH = 224
WW = 224
NPIX = HH * WW
RCH = 16
NC = RCH * WW
TCH = HH // RCH
HALO = 512
NHB = NPIX // HALO
SUMCH = 1024
NSUM = NPIX // SUMCH


def _sums_kernel(x_ref, o_ref):
    @pl.when(pl.program_id(0) == 0)
    def _init():
        o_ref[...] = jnp.zeros_like(o_ref)

    o_ref[...] += jnp.sum(x_ref[...], axis=1, keepdims=True)


def _gate_kernel(sums_ref, wg_ref, s_ref, loss_ref):
    gx = sums_ref[...] * (1.0 / NPIX)
    logits = jnp.dot(gx, wg_ref[...],
                     preferred_element_type=jnp.float32)
    m = jnp.max(logits, axis=1, keepdims=True)
    ex = jnp.exp(logits - m)
    p = ex / jnp.sum(ex, axis=1, keepdims=True)
    lane = jax.lax.broadcasted_iota(jnp.int32, p.shape, 1)
    v1 = jnp.max(p, axis=1, keepdims=True)
    i1 = jnp.min(jnp.where(p == v1, lane, EN), axis=1, keepdims=True)
    m1 = lane == i1
    pm = jnp.where(m1, -jnp.inf, p)
    v2 = jnp.max(pm, axis=1, keepdims=True)
    i2 = jnp.min(jnp.where(pm == v2, lane, EN), axis=1, keepdims=True)
    m2 = lane == i2
    tot = v1 + v2
    denom = tot + 1e-6
    gates = (jnp.where(m1, v1, 0.0) + jnp.where(m2, v2, 0.0)) / denom
    s_ref[...] = tot / denom
    imp = jnp.sum(gates, axis=0, keepdims=True)
    load = jnp.sum((gates > 0.0).astype(jnp.float32), axis=0, keepdims=True)

    def cv2(v):
        mu = jnp.sum(v, axis=1, keepdims=True) / EN
        var = jnp.sum((v - mu) ** 2, axis=1, keepdims=True) / (EN - 1)
        return var / (mu * mu + 1e-10)

    loss_ref[...] = (cv2(imp) + cv2(load)) * 0.01


def _conv_kernel(cb_ref, s_ref, xp_ref, xc_ref, xn_ref, wt_ref, o_ref):
    i = pl.program_id(1)
    c2 = cb_ref[:, 0:1]
    c3 = cb_ref[:, 1:2]
    xall = jnp.concatenate(
        [xp_ref[0][:, HALO - WW:], xc_ref[0], xn_ref[0][:, :WW]],
        axis=1)
    L = NC + 2 * WW
    t = jnp.tanh(xall)
    p2 = t * t - c2
    p3 = t * (p2 - c3)
    gi = jnp.concatenate([t, p2, p3], axis=0)
    gi = gi * jax.nn.sigmoid(gi)
    lane = jax.lax.broadcasted_iota(jnp.int32, (1, L), 1)
    gpos = (i * RCH - 1) * WW + lane
    valid = (gpos >= 0) & (gpos < NPIX)
    c0 = 1.0 / (1.0 + 2.718281828459045 ** -1.0)
    g = jnp.concatenate(
        [jnp.broadcast_to(jnp.float32(c0), (C, L)), gi], axis=0)
    g = jnp.where(valid, g, 0.0).astype(jnp.bfloat16)
    col = lane % WW
    zero = jnp.bfloat16(0.0)
    gm = jnp.where(col == 0, zero, jnp.roll(g, 1, axis=1))
    gp = jnp.where(col == WW - 1, zero, jnp.roll(g, -1, axis=1))
    acc = jnp.zeros((C, NC), jnp.float32)
    for ky in range(3):
        base = ky * WW
        for kx, gg in ((0, gm), (1, g), (2, gp)):
            w = wt_ref[ky, kx]
            acc = acc + jax.lax.dot_general(
                w, gg[:, base:base + NC],
                (((1,), (0,)), ((), ())),
                preferred_element_type=jnp.float32)
    o_ref[0] = acc * s_ref[pl.ds(pl.program_id(0), 1), :]


def _betac(n, m, bw):
    return (m + n) * (m - n) * n ** 2 / (m ** 2 / (4.0 * n ** 2 - 1.0)) * bw[n]


def kernel(x, poly_weights, beta_weights, w_gate):
    x = x.astype(jnp.float32)
    xflat = x.reshape(B, C, NPIX)
    sums = pl.pallas_call(
        _sums_kernel,
        grid=(NSUM,),
        in_specs=[pl.BlockSpec((B * C, SUMCH), lambda i: (0, i))],
        out_specs=pl.BlockSpec((B * C, 1), lambda i: (0, 0)),
        out_shape=jax.ShapeDtypeStruct((B * C, 1), jnp.float32),
    )(x.reshape(B * C, NPIX))

    s, loss = pl.pallas_call(
        _gate_kernel,
        out_shape=(
            jax.ShapeDtypeStruct((B, 1), jnp.float32),
            jax.ShapeDtypeStruct((1, 1), jnp.float32),
        ),
    )(sums.reshape(B, C), w_gate)

    wt = jnp.transpose(poly_weights[0], (2, 3, 0, 1)).astype(jnp.bfloat16)
    cb = jnp.stack([_betac(1, 2, beta_weights),
                    _betac(2, 3, beta_weights)]).reshape(1, 2)

    yflat = pl.pallas_call(
        _conv_kernel,
        grid=(B, TCH),
        in_specs=[
            pl.BlockSpec((1, 2), lambda b, i: (0, 0)),
            pl.BlockSpec((B, 1), lambda b, i: (0, 0)),
            pl.BlockSpec((1, C, HALO),
                         lambda b, i: (b, 0, jnp.maximum(i * (NC // HALO) - 1, 0))),
            pl.BlockSpec((1, C, NC), lambda b, i: (b, 0, i)),
            pl.BlockSpec((1, C, HALO),
                         lambda b, i: (b, 0, jnp.minimum((i + 1) * (NC // HALO),
                                                         NHB - 1))),
            pl.BlockSpec((3, 3, C, 4 * C), lambda b, i: (0, 0, 0, 0)),
        ],
        out_specs=pl.BlockSpec((1, C, NC), lambda b, i: (b, 0, i)),
        out_shape=jax.ShapeDtypeStruct((B, C, NPIX), jnp.float32),
    )(cb, s, xflat, xflat, xflat, wt)

    return (yflat.reshape(B, C, HH, WW), jnp.reshape(loss, ()))


# speedup vs baseline: 1.1148x; 1.1148x over previous
import jax
import jax.numpy as jnp
from jax.experimental import pallas as pl

B = 2
C = 96
EN = 8
HH = 224
WW = 224
NPIX = HH * WW
RCH = 28
NC = RCH * WW
TCH = HH // RCH
HALO = 896
NHB = NPIX // HALO
SUMCH = 3584
NSUM = NPIX // SUMCH


def _sums_kernel(x_ref, o_ref):
    @pl.when(pl.program_id(0) == 0)
    def _init():
        o_ref[...] = jnp.zeros_like(o_ref)

    o_ref[...] += jnp.sum(x_ref[...], axis=1, keepdims=True)


def _gate_kernel(sums_ref, wg_ref, s_ref, loss_ref):
    gx = sums_ref[...] * (1.0 / NPIX)
    logits = jnp.dot(gx, wg_ref[...],
                     preferred_element_type=jnp.float32)
    m = jnp.max(logits, axis=1, keepdims=True)
    ex = jnp.exp(logits - m)
    p = ex / jnp.sum(ex, axis=1, keepdims=True)
    lane = jax.lax.broadcasted_iota(jnp.int32, p.shape, 1)
    v1 = jnp.max(p, axis=1, keepdims=True)
    i1 = jnp.min(jnp.where(p == v1, lane, EN), axis=1, keepdims=True)
    m1 = lane == i1
    pm = jnp.where(m1, -jnp.inf, p)
    v2 = jnp.max(pm, axis=1, keepdims=True)
    i2 = jnp.min(jnp.where(pm == v2, lane, EN), axis=1, keepdims=True)
    m2 = lane == i2
    tot = v1 + v2
    denom = tot + 1e-6
    gates = (jnp.where(m1, v1, 0.0) + jnp.where(m2, v2, 0.0)) / denom
    s_ref[...] = tot / denom
    imp = jnp.sum(gates, axis=0, keepdims=True)
    load = jnp.sum((gates > 0.0).astype(jnp.float32), axis=0, keepdims=True)

    def cv2(v):
        mu = jnp.sum(v, axis=1, keepdims=True) / EN
        var = jnp.sum((v - mu) ** 2, axis=1, keepdims=True) / (EN - 1)
        return var / (mu * mu + 1e-10)

    loss_ref[...] = (cv2(imp) + cv2(load)) * 0.01


def _conv_kernel(cb_ref, s_ref, xp_ref, xc_ref, xn_ref, wt_ref, o_ref):
    i = pl.program_id(1)
    c2 = cb_ref[:, 0:1]
    c3 = cb_ref[:, 1:2]
    xall = jnp.concatenate(
        [xp_ref[0][:, HALO - WW:], xc_ref[0], xn_ref[0][:, :WW]],
        axis=1)
    L = NC + 2 * WW
    t = jnp.tanh(xall)
    p2 = t * t - c2
    p3 = t * (p2 - c3)
    gi = jnp.concatenate([t, p2, p3], axis=0)
    gi = gi * jax.nn.sigmoid(gi)
    lane = jax.lax.broadcasted_iota(jnp.int32, (1, L), 1)
    gpos = (i * RCH - 1) * WW + lane
    valid = (gpos >= 0) & (gpos < NPIX)
    c0 = 1.0 / (1.0 + 2.718281828459045 ** -1.0)
    g = jnp.concatenate(
        [jnp.broadcast_to(jnp.float32(c0), (C, L)), gi], axis=0)
    g = g.astype(jnp.bfloat16) * valid.astype(jnp.bfloat16)
    col = lane % WW
    zero = jnp.bfloat16(0.0)
    gm = jnp.where(col == 0, zero, jnp.roll(g, 1, axis=1))
    gp = jnp.where(col == WW - 1, zero, jnp.roll(g, -1, axis=1))
    acc = jnp.zeros((C, NC), jnp.float32)
    for ky in range(3):
        base = ky * WW
        for kx, gg in ((0, gm), (1, g), (2, gp)):
            w = wt_ref[ky, kx]
            acc = acc + jax.lax.dot_general(
                w, gg[:, base:base + NC],
                (((1,), (0,)), ((), ())),
                preferred_element_type=jnp.float32)
    o_ref[0] = acc * s_ref[pl.ds(pl.program_id(0), 1), :]


def _betac(n, m, bw):
    return (m + n) * (m - n) * n ** 2 / (m ** 2 / (4.0 * n ** 2 - 1.0)) * bw[n]


def kernel(x, poly_weights, beta_weights, w_gate):
    x = x.astype(jnp.float32)
    xflat = x.reshape(B, C, NPIX)
    sums = pl.pallas_call(
        _sums_kernel,
        grid=(NSUM,),
        in_specs=[pl.BlockSpec((B * C, SUMCH), lambda i: (0, i))],
        out_specs=pl.BlockSpec((B * C, 1), lambda i: (0, 0)),
        out_shape=jax.ShapeDtypeStruct((B * C, 1), jnp.float32),
    )(x.reshape(B * C, NPIX))

    s, loss = pl.pallas_call(
        _gate_kernel,
        out_shape=(
            jax.ShapeDtypeStruct((B, 1), jnp.float32),
            jax.ShapeDtypeStruct((1, 1), jnp.float32),
        ),
    )(sums.reshape(B, C), w_gate)

    wt = jnp.transpose(poly_weights[0], (2, 3, 0, 1)).astype(jnp.bfloat16)
    cb = jnp.stack([_betac(1, 2, beta_weights),
                    _betac(2, 3, beta_weights)]).reshape(1, 2)

    yflat = pl.pallas_call(
        _conv_kernel,
        grid=(B, TCH),
        in_specs=[
            pl.BlockSpec((1, 2), lambda b, i: (0, 0)),
            pl.BlockSpec((B, 1), lambda b, i: (0, 0)),
            pl.BlockSpec((1, C, HALO),
                         lambda b, i: (b, 0, jnp.maximum(i * (NC // HALO) - 1, 0))),
            pl.BlockSpec((1, C, NC), lambda b, i: (b, 0, i)),
            pl.BlockSpec((1, C, HALO),
                         lambda b, i: (b, 0, jnp.minimum((i + 1) * (NC // HALO),
                                                         NHB - 1))),
            pl.BlockSpec((3, 3, C, 4 * C), lambda b, i: (0, 0, 0, 0)),
        ],
        out_specs=pl.BlockSpec((1, C, NC), lambda b, i: (b, 0, i)),
        out_shape=jax.ShapeDtypeStruct((B, C, NPIX), jnp.float32),
    )(cb, s, xflat, xflat, xflat, wt)

    return (yflat.reshape(B, C, HH, WW), jnp.reshape(loss, ()))
